# Initial kernel scaffold; baseline (speedup 1.0000x reference)
import jax
import jax.numpy as jnp
from jax.experimental import pallas as pl


def kernel(context, query, W, b):
    raise NotImplementedError("write your pallas kernel here")


# speedup vs baseline: 1.1096x; 1.1096x over previous
import jax
import jax.numpy as jnp
import numpy as np
from jax.experimental import pallas as pl
from jax.experimental.pallas import tpu as pltpu

N_ACT = 100000
D_IN = 128
BATCH = 1024
B_TILE = 256
A_TILE = 2048
NB = BATCH // B_TILE
NA = (N_ACT + A_TILE - 1) // A_TILE

_NEG_INF = np.float32(-np.inf)
_TINY = np.float32(np.finfo(np.float32).tiny)


def _threefry_bits(i):
    u32 = lambda v: jnp.uint32(v)
    ks0 = u32(0)
    ks1 = u32(42)
    ks2 = u32(0 ^ 42 ^ 0x1BD11BDA)
    x0 = jnp.zeros_like(i) + ks0
    x1 = i + ks1

    def rotl(x, d):
        return (x << u32(d)) | (x >> u32(32 - d))

    def rounds(x0, x1, rots):
        for r in rots:
            x0 = x0 + x1
            x1 = rotl(x1, r)
            x1 = x0 ^ x1
        return x0, x1

    r_even = (13, 15, 26, 6)
    r_odd = (17, 29, 16, 24)
    x0, x1 = rounds(x0, x1, r_even)
    x0 = x0 + ks1
    x1 = x1 + ks2 + u32(1)
    x0, x1 = rounds(x0, x1, r_odd)
    x0 = x0 + ks2
    x1 = x1 + ks0 + u32(2)
    x0, x1 = rounds(x0, x1, r_even)
    x0 = x0 + ks0
    x1 = x1 + ks1 + u32(3)
    x0, x1 = rounds(x0, x1, r_odd)
    x0 = x0 + ks1
    x1 = x1 + ks2 + u32(4)
    x0, x1 = rounds(x0, x1, r_even)
    x0 = x0 + ks2
    x1 = x1 + ks0 + u32(5)
    return x0 ^ x1


def _policy_kernel(x_ref, w_ref, b_ref, idx_ref, prob_ref,
                   best_z, best_idx, best_l, m_l, s_sum):
    a = pl.program_id(1)

    @pl.when(a == 0)
    def _init():
        best_z[...] = jnp.full((B_TILE, 1), _NEG_INF, jnp.float32)
        best_idx[...] = jnp.zeros((B_TILE, 1), jnp.int32)
        best_l[...] = jnp.full((B_TILE, 1), _NEG_INF, jnp.float32)
        m_l[...] = jnp.full((B_TILE, 1), _NEG_INF, jnp.float32)
        s_sum[...] = jnp.zeros((B_TILE, 1), jnp.float32)

    l = jnp.dot(x_ref[...], w_ref[...],
                preferred_element_type=jnp.float32,
                precision=jax.lax.Precision.HIGHEST) + b_ref[...]

    r0 = pl.program_id(0) * B_TILE
    a0 = a * A_TILE
    row_u = jax.lax.broadcasted_iota(jnp.uint32, (B_TILE, A_TILE), 0)
    col_i = jax.lax.broadcasted_iota(jnp.int32, (B_TILE, A_TILE), 1)
    cnt = (row_u + jnp.uint32(r0)) * jnp.uint32(N_ACT) \
        + col_i.astype(jnp.uint32) + jnp.uint32(a0)
    bits = _threefry_bits(cnt)
    fb = (bits >> jnp.uint32(9)) | jnp.uint32(0x3F800000)
    u = jax.lax.bitcast_convert_type(fb, jnp.float32) - jnp.float32(1.0)
    u = jnp.maximum(u, _TINY)
    g = -jnp.log(-jnp.log(u))
    z = l + g

    valid = (col_i + a0) < N_ACT
    zm = jnp.where(valid, z, _NEG_INF)
    lm = jnp.where(valid, l, _NEG_INF)

    t_max = jnp.max(zm, axis=1, keepdims=True)
    cand = jnp.where(zm == t_max, col_i, jnp.int32(2**31 - 1))
    t_arg = jnp.min(cand, axis=1, keepdims=True)
    l_at = jnp.max(jnp.where(col_i == t_arg, l, _NEG_INF), axis=1,
                   keepdims=True)

    upd = t_max > best_z[...]
    best_idx[...] = jnp.where(upd, t_arg + a0, best_idx[...])
    best_l[...] = jnp.where(upd, l_at, best_l[...])
    best_z[...] = jnp.maximum(best_z[...], t_max)

    t_ml = jnp.max(lm, axis=1, keepdims=True)
    new_m = jnp.maximum(m_l[...], t_ml)
    s_sum[...] = s_sum[...] * jnp.exp(m_l[...] - new_m) \
        + jnp.sum(jnp.exp(lm - new_m), axis=1, keepdims=True)
    m_l[...] = new_m

    @pl.when(a == NA - 1)
    def _done():
        idx_ref[...] = best_idx[...]
        prob_ref[...] = jnp.exp(best_l[...] - m_l[...]
                                - jnp.log(s_sum[...]))


def _pallas_specs():
    return dict(
        grid=(NB, NA),
        in_specs=[
            pl.BlockSpec((B_TILE, D_IN), lambda i, j: (i, 0)),
            pl.BlockSpec((D_IN, A_TILE), lambda i, j: (0, j)),
            pl.BlockSpec((1, A_TILE), lambda i, j: (0, j)),
        ],
        out_specs=[
            pl.BlockSpec((B_TILE, 1), lambda i, j: (i, 0)),
            pl.BlockSpec((B_TILE, 1), lambda i, j: (i, 0)),
        ],
        out_shape=[
            jax.ShapeDtypeStruct((BATCH, 1), jnp.int32),
            jax.ShapeDtypeStruct((BATCH, 1), jnp.float32),
        ],
        scratch_shapes=[
            pltpu.VMEM((B_TILE, 1), jnp.float32),
            pltpu.VMEM((B_TILE, 1), jnp.int32),
            pltpu.VMEM((B_TILE, 1), jnp.float32),
            pltpu.VMEM((B_TILE, 1), jnp.float32),
            pltpu.VMEM((B_TILE, 1), jnp.float32),
        ],
        compiler_params=pltpu.CompilerParams(
            dimension_semantics=("parallel", "arbitrary")),
    )


@jax.jit
def _run(inputs, W, b2d):
    idx2d, prob2d = pl.pallas_call(_policy_kernel, **_pallas_specs())(
        inputs, W, b2d)
    return idx2d[:, 0], prob2d[:, 0]


def kernel(context, query, W, b):
    inputs = jnp.concatenate((context, query), axis=1)
    return _run(inputs, W, b.reshape(1, N_ACT))


# speedup vs baseline: 1.2533x; 1.1296x over previous
import jax
import jax.numpy as jnp
import numpy as np
from jax.experimental import pallas as pl
from jax.experimental.pallas import tpu as pltpu

N_ACT = 100000
D_IN = 128
BATCH = 1024
B_TILE = 256
A_TILE = 2048
NB = BATCH // B_TILE
NA = (N_ACT + A_TILE - 1) // A_TILE

_NEG_INF = np.float32(-np.inf)
_TINY = np.float32(np.finfo(np.float32).tiny)
_INT_MAX = np.int32(2**31 - 1)


def _threefry_bits(i):
    u32 = lambda v: jnp.uint32(v)
    ks0 = u32(0)
    ks1 = u32(42)
    ks2 = u32(0 ^ 42 ^ 0x1BD11BDA)

    def rotl(x, d):
        return (x << u32(d)) | (x >> u32(32 - d))

    def rounds(x0, x1, rots):
        for r in rots:
            x0 = x0 + x1
            x1 = rotl(x1, r)
            x1 = x0 ^ x1
        return x0, x1

    r_even = (13, 15, 26, 6)
    r_odd = (17, 29, 16, 24)

    x1 = i + ks1
    x0 = x1
    x1 = x0 ^ rotl(x1, 13)
    x0, x1 = rounds(x0, x1, r_even[1:])
    x0 = x0 + ks1
    x1 = x1 + ks2 + u32(1)
    x0, x1 = rounds(x0, x1, r_odd)
    x0 = x0 + ks2
    x1 = x1 + ks0 + u32(2)
    x0, x1 = rounds(x0, x1, r_even)
    x0 = x0 + ks0
    x1 = x1 + ks1 + u32(3)
    x0, x1 = rounds(x0, x1, r_odd)
    x0 = x0 + ks1
    x1 = x1 + ks2 + u32(4)
    x0, x1 = rounds(x0, x1, r_even)
    x0 = x0 + ks2
    x1 = x1 + ks0 + u32(5)
    return x0 ^ x1


def _gumbel(cnt_u32):
    bits = _threefry_bits(cnt_u32)
    fb = (bits >> jnp.uint32(9)) | jnp.uint32(0x3F800000)
    u = jax.lax.bitcast_convert_type(fb, jnp.float32) - jnp.float32(1.0)
    u = jnp.maximum(u, _TINY)
    return -jnp.log(-jnp.log(u))


def _policy_kernel(x_ref, w_ref, b_ref, idx_ref, prob_ref,
                   best_z, best_cnt, m_l, s_sum, cnt_base):
    a = pl.program_id(1)
    r0 = pl.program_id(0) * B_TILE

    @pl.when(a == 0)
    def _init():
        best_z[...] = jnp.full((B_TILE, 1), _NEG_INF, jnp.float32)
        best_cnt[...] = jnp.zeros((B_TILE, 1), jnp.int32)
        m_l[...] = jnp.full((B_TILE, 1), _NEG_INF, jnp.float32)
        s_sum[...] = jnp.zeros((B_TILE, 1), jnp.float32)
        row = jax.lax.broadcasted_iota(jnp.int32, (B_TILE, A_TILE), 0)
        col = jax.lax.broadcasted_iota(jnp.int32, (B_TILE, A_TILE), 1)
        cnt_base[...] = (row + r0) * jnp.int32(N_ACT) + col

    def _step(masked):
        l = jnp.dot(x_ref[...], w_ref[...],
                    preferred_element_type=jnp.float32) + b_ref[...]
        cnt = cnt_base[...] + a * A_TILE
        g = _gumbel(cnt.astype(jnp.uint32))
        z = l + g
        if masked:
            valid = cnt - cnt_base[...][:, :1] < N_ACT
            z = jnp.where(valid, z, _NEG_INF)
            l = jnp.where(valid, l, _NEG_INF)

        t_max = jnp.max(z, axis=1, keepdims=True)
        cand = jnp.where(z == t_max, cnt, _INT_MAX)
        t_cnt = jnp.min(cand, axis=1, keepdims=True)
        upd = t_max > best_z[...]
        best_cnt[...] = jnp.where(upd, t_cnt, best_cnt[...])
        best_z[...] = jnp.maximum(best_z[...], t_max)

        t_ml = jnp.max(l, axis=1, keepdims=True)
        new_m = jnp.maximum(m_l[...], t_ml)
        s_sum[...] = s_sum[...] * jnp.exp(m_l[...] - new_m) \
            + jnp.sum(jnp.exp(l - new_m), axis=1, keepdims=True)
        m_l[...] = new_m

    @pl.when(a < NA - 1)
    def _main():
        _step(masked=False)

    @pl.when(a == NA - 1)
    def _last():
        _step(masked=True)
        g_best = _gumbel(best_cnt[...].astype(jnp.uint32))
        l_best = best_z[...] - g_best
        row_base = cnt_base[...][:, :1]
        idx_ref[...] = best_cnt[...] - row_base
        prob_ref[...] = jnp.exp(l_best - m_l[...] - jnp.log(s_sum[...]))


def _pallas_specs():
    return dict(
        grid=(NB, NA),
        in_specs=[
            pl.BlockSpec((B_TILE, D_IN), lambda i, j: (i, 0)),
            pl.BlockSpec((D_IN, A_TILE), lambda i, j: (0, j)),
            pl.BlockSpec((1, A_TILE), lambda i, j: (0, j)),
        ],
        out_specs=[
            pl.BlockSpec((B_TILE, 1), lambda i, j: (i, 0)),
            pl.BlockSpec((B_TILE, 1), lambda i, j: (i, 0)),
        ],
        out_shape=[
            jax.ShapeDtypeStruct((BATCH, 1), jnp.int32),
            jax.ShapeDtypeStruct((BATCH, 1), jnp.float32),
        ],
        scratch_shapes=[
            pltpu.VMEM((B_TILE, 1), jnp.float32),
            pltpu.VMEM((B_TILE, 1), jnp.int32),
            pltpu.VMEM((B_TILE, 1), jnp.float32),
            pltpu.VMEM((B_TILE, 1), jnp.float32),
            pltpu.VMEM((B_TILE, A_TILE), jnp.int32),
        ],
        compiler_params=pltpu.CompilerParams(
            dimension_semantics=("parallel", "arbitrary")),
    )


@jax.jit
def _run(inputs, W, b2d):
    idx2d, prob2d = pl.pallas_call(_policy_kernel, **_pallas_specs())(
        inputs, W, b2d)
    return idx2d[:, 0], prob2d[:, 0]


def kernel(context, query, W, b):
    inputs = jnp.concatenate((context, query), axis=1)
    return _run(inputs, W, b.reshape(1, N_ACT))


# speedup vs baseline: 1.2821x; 1.0230x over previous
import jax
import jax.numpy as jnp
import numpy as np
from jax.experimental import pallas as pl
from jax.experimental.pallas import tpu as pltpu

N_ACT = 100000
D_IN = 128
BATCH = 1024
B_TILE = 256
A_TILE = 2048
NB = BATCH // B_TILE
NA = (N_ACT + A_TILE - 1) // A_TILE

_NEG_INF = np.float32(-np.inf)
_TINY = np.float32(np.finfo(np.float32).tiny)
_INT_MAX = np.int32(2**31 - 1)


def _threefry_bits(i):
    u32 = lambda v: jnp.uint32(v)
    ks0 = u32(0)
    ks1 = u32(42)
    ks2 = u32(0 ^ 42 ^ 0x1BD11BDA)

    def rotl(x, d):
        return (x << u32(d)) | (x >> u32(32 - d))

    def rounds(x0, x1, rots):
        for r in rots:
            x0 = x0 + x1
            x1 = rotl(x1, r)
            x1 = x0 ^ x1
        return x0, x1

    r_even = (13, 15, 26, 6)
    r_odd = (17, 29, 16, 24)

    x1 = i
    x0 = x1
    x1 = x0 ^ rotl(x1, 13)
    x0, x1 = rounds(x0, x1, r_even[1:])
    x0 = x0 + ks1
    x1 = x1 + ks2 + u32(1)
    x0, x1 = rounds(x0, x1, r_odd)
    x0 = x0 + ks2
    x1 = x1 + ks0 + u32(2)
    x0, x1 = rounds(x0, x1, r_even)
    x0 = x0 + ks0
    x1 = x1 + ks1 + u32(3)
    x0, x1 = rounds(x0, x1, r_odd)
    x0 = x0 + ks1
    x1 = x1 + ks2 + u32(4)
    x0, x1 = rounds(x0, x1, r_even)
    x0 = x0 + ks2
    x1 = x1 + ks0 + u32(5)
    return x0 ^ x1


def _neg_gumbel(cnt_plus_42):
    bits = _threefry_bits(cnt_plus_42)
    fb = (bits >> jnp.uint32(9)) | jnp.uint32(0x3F800000)
    u = jax.lax.bitcast_convert_type(fb, jnp.float32) - jnp.float32(1.0)
    u = jnp.maximum(u, _TINY)
    return jnp.log(-jnp.log(u))


def _policy_kernel(x_ref, w_ref, b_ref, idx_ref, prob_ref,
                   best_z, best_cnt, s_sum, cnt_base):
    a = pl.program_id(1)
    r0 = pl.program_id(0) * B_TILE

    @pl.when(a == 0)
    def _init():
        best_z[...] = jnp.full((B_TILE, 1), _NEG_INF, jnp.float32)
        best_cnt[...] = jnp.zeros((B_TILE, 1), jnp.int32)
        s_sum[...] = jnp.zeros((B_TILE, 1), jnp.float32)
        row = jax.lax.broadcasted_iota(jnp.int32, (B_TILE, A_TILE), 0)
        col = jax.lax.broadcasted_iota(jnp.int32, (B_TILE, A_TILE), 1)
        cnt_base[...] = (row + r0) * jnp.int32(N_ACT) + col

    def _step(masked):
        l = jnp.dot(x_ref[...], w_ref[...],
                    preferred_element_type=jnp.float32) + b_ref[...]
        ng = _neg_gumbel((cnt_base[...] + (a * A_TILE + 42)).astype(jnp.uint32))
        z = l - ng
        if masked:
            col = cnt_base[...] - cnt_base[...][:, :1]
            valid = col < N_ACT - a * A_TILE
            z = jnp.where(valid, z, _NEG_INF)
            l = jnp.where(valid, l, _NEG_INF)

        t_max = jnp.max(z, axis=1, keepdims=True)
        cand = jnp.where(z == t_max, cnt_base[...], _INT_MAX)
        t_cnt = jnp.min(cand, axis=1, keepdims=True) + a * A_TILE
        upd = t_max > best_z[...]
        best_cnt[...] = jnp.where(upd, t_cnt, best_cnt[...])

        m_old = best_z[...]
        m_new = jnp.maximum(m_old, t_max)
        s_sum[...] = s_sum[...] * jnp.exp(m_old - m_new) \
            + jnp.sum(jnp.exp(l - m_new), axis=1, keepdims=True)
        best_z[...] = m_new

    @pl.when(a < NA - 1)
    def _main():
        _step(masked=False)

    @pl.when(a == NA - 1)
    def _last():
        _step(masked=True)
        ng_best = _neg_gumbel((best_cnt[...] + 42).astype(jnp.uint32))
        row_base = cnt_base[...][:, :1]
        idx_ref[...] = best_cnt[...] - row_base
        prob_ref[...] = jnp.exp(ng_best - jnp.log(s_sum[...]))


def _pallas_specs():
    return dict(
        grid=(NB, NA),
        in_specs=[
            pl.BlockSpec((B_TILE, D_IN), lambda i, j: (i, 0)),
            pl.BlockSpec((D_IN, A_TILE), lambda i, j: (0, j)),
            pl.BlockSpec((1, A_TILE), lambda i, j: (0, j)),
        ],
        out_specs=[
            pl.BlockSpec((B_TILE, 1), lambda i, j: (i, 0)),
            pl.BlockSpec((B_TILE, 1), lambda i, j: (i, 0)),
        ],
        out_shape=[
            jax.ShapeDtypeStruct((BATCH, 1), jnp.int32),
            jax.ShapeDtypeStruct((BATCH, 1), jnp.float32),
        ],
        scratch_shapes=[
            pltpu.VMEM((B_TILE, 1), jnp.float32),
            pltpu.VMEM((B_TILE, 1), jnp.int32),
            pltpu.VMEM((B_TILE, 1), jnp.float32),
            pltpu.VMEM((B_TILE, A_TILE), jnp.int32),
        ],
        compiler_params=pltpu.CompilerParams(
            dimension_semantics=("parallel", "arbitrary")),
    )


@jax.jit
def _run(inputs, W, b2d):
    idx2d, prob2d = pl.pallas_call(_policy_kernel, **_pallas_specs())(
        inputs, W, b2d)
    return idx2d[:, 0], prob2d[:, 0]


def kernel(context, query, W, b):
    inputs = jnp.concatenate((context, query), axis=1)
    return _run(inputs, W, b.reshape(1, N_ACT))
